# baseline (speedup 1.0000x reference)
import functools

import jax
import jax.numpy as jnp
from jax import lax
from jax.experimental import pallas as pl
from jax.experimental.pallas import tpu as pltpu
from jax.experimental.pallas import tpu_sc as plsc

_B, _D, _O, _E = 256, 768, 256, 16
_EB = 8

_NC, _NS, _L = 2, 16, 16
_NW = _NC * _NS
_TPW = _B // _NW


def _tc_dense_kernel(x_ref, rw_ref, brw_ref, w_ref, y_ref, lw_ref, lb_ref):
    g = pl.program_id(0)

    @pl.when(g == 0)
    def _logits():
        x = x_ref[...]
        lw_ref[...] = jnp.dot(x, rw_ref[...], preferred_element_type=jnp.float32)
        lb_ref[...] = jnp.dot(x, brw_ref[...], preferred_element_type=jnp.float32)

    xb = x_ref[...].astype(jnp.bfloat16)
    ys = [
        jnp.dot(xb, w_ref[j].astype(jnp.bfloat16),
                preferred_element_type=jnp.float32)
        for j in range(_EB)
    ]
    y_ref[...] = jnp.stack(ys, axis=1)


def _tc_dense(x, rw, brw, wbank):
    return pl.pallas_call(
        _tc_dense_kernel,
        grid=(_E // _EB,),
        in_specs=[
            pl.BlockSpec((_B, _D), lambda g: (0, 0)),
            pl.BlockSpec((_D, _E), lambda g: (0, 0)),
            pl.BlockSpec((_D, _E), lambda g: (0, 0)),
            pl.BlockSpec((_EB, _D, _O), lambda g: (g, 0, 0)),
        ],
        out_specs=[
            pl.BlockSpec((_B, _EB, _O), lambda g: (0, g, 0)),
            pl.BlockSpec((_B, _E), lambda g: (0, 0)),
            pl.BlockSpec((_B, _E), lambda g: (0, 0)),
        ],
        out_shape=[
            jax.ShapeDtypeStruct((_B, _E, _O), jnp.float32),
            jax.ShapeDtypeStruct((_B, _E), jnp.float32),
            jax.ShapeDtypeStruct((_B, _E), jnp.float32),
        ],
        compiler_params=pltpu.CompilerParams(
            dimension_semantics=("arbitrary",),
        ),
    )(x, rw, brw, wbank)


def _top2(lv):
    m1 = jnp.float32(-jnp.inf)
    m2 = jnp.float32(-jnp.inf)
    i1 = jnp.int32(0)
    i2 = jnp.int32(0)
    for i in range(_E):
        l = lv[i]
        gt1 = l > m1
        gt2 = l > m2
        m2 = jnp.where(gt1, m1, jnp.where(gt2, l, m2))
        i2 = jnp.where(gt1, i1, jnp.where(gt2, jnp.int32(i), i2))
        m1 = jnp.where(gt1, l, m1)
        i1 = jnp.where(gt1, jnp.int32(i), i1)
    d = jnp.full((_L,), m2 - m1, jnp.float32)
    p1v = 1.0 / (1.0 + jnp.exp(d))
    p2v = 1.0 - p1v
    return p1v, p2v, i1, i2


def _sc_combine_kernel(lw_hbm, lb_hbm, y_hbm, bb_hbm, out_hbm,
                       lw_v, lb_v, yrows_v, brows_v, outbuf_v, sem):
    wid = lax.axis_index("s") * _NC + lax.axis_index("c")
    base = wid * _TPW

    pltpu.sync_copy(lw_hbm.at[pl.ds(base, _TPW)], lw_v)
    pltpu.sync_copy(lb_hbm.at[pl.ds(base, _TPW)], lb_v)

    iota = lax.iota(jnp.int32, _L)
    yidx = jnp.zeros((_L,), jnp.int32)
    bidx = jnp.zeros((_L,), jnp.int32)
    pw = []
    qb = []
    for t in range(_TPW):
        p1v, p2v, i1, i2 = _top2(lw_v[t, :])
        q1v, q2v, j1, j2 = _top2(lb_v[t, :])
        tok = base + t
        yidx = jnp.where(iota == 2 * t, tok * _E + i1, yidx)
        yidx = jnp.where(iota == 2 * t + 1, tok * _E + i2, yidx)
        bidx = jnp.where(iota == 2 * t, j1, bidx)
        bidx = jnp.where(iota == 2 * t + 1, j2, bidx)
        pw.append((p1v, p2v))
        qb.append((q1v, q2v))

    pltpu.async_copy(y_hbm.at[yidx], yrows_v, sem).wait()
    pltpu.async_copy(bb_hbm.at[bidx], brows_v, sem).wait()

    for t in range(_TPW):
        p1v, p2v = pw[t]
        q1v, q2v = qb[t]
        for s in range(_O // _L):
            sl = pl.ds(s * _L, _L)
            acc = (p1v * yrows_v[2 * t, sl] + p2v * yrows_v[2 * t + 1, sl]
                   + q1v * brows_v[2 * t, sl] + q2v * brows_v[2 * t + 1, sl])
            outbuf_v[t, sl] = acc

    pltpu.sync_copy(outbuf_v, out_hbm.at[pl.ds(base, _TPW)])


@functools.partial(
    pl.kernel,
    mesh=plsc.VectorSubcoreMesh(core_axis_name="c", subcore_axis_name="s"),
    out_type=jax.ShapeDtypeStruct((_B, _O), jnp.float32),
    scratch_types=[
        pltpu.VMEM((_TPW, _E), jnp.float32),
        pltpu.VMEM((_TPW, _E), jnp.float32),
        pltpu.VMEM((2 * _TPW, _O), jnp.float32),
        pltpu.VMEM((2 * _TPW, _O), jnp.float32),
        pltpu.VMEM((_TPW, _O), jnp.float32),
        pltpu.SemaphoreType.DMA,
    ],
)
def _sc_combine(lw_hbm, lb_hbm, y_hbm, bb_hbm, out_hbm,
                lw_v, lb_v, yrows_v, brows_v, outbuf_v, sem):
    _sc_combine_kernel(lw_hbm, lb_hbm, y_hbm, bb_hbm, out_hbm,
                       lw_v, lb_v, yrows_v, brows_v, outbuf_v, sem)


def kernel(input_batch, router_w, bias_router_w, weight_bank, bias_bank):
    y, lw, lb = _tc_dense(input_batch, router_w, bias_router_w, weight_bank)
    y2 = y.reshape(_B * _E, _O)
    return _sc_combine(lw, lb, y2, bias_bank)


# speedup vs baseline: 3.2517x; 3.2517x over previous
import jax
import jax.numpy as jnp
from jax.experimental import pallas as pl
from jax.experimental.pallas import tpu as pltpu

_B, _D, _O, _E = 256, 768, 256, 16


def _topk2_coeffs(logits):
    iota = jax.lax.broadcasted_iota(jnp.int32, logits.shape, 1)
    l1 = jnp.max(logits, axis=-1, keepdims=True)
    i1 = jnp.min(jnp.where(logits == l1, iota, _E), axis=-1, keepdims=True)
    masked = jnp.where(iota == i1, -jnp.inf, logits)
    l2 = jnp.max(masked, axis=-1, keepdims=True)
    i2 = jnp.min(jnp.where(masked == l2, iota, _E), axis=-1, keepdims=True)
    p1 = 1.0 / (1.0 + jnp.exp(l2 - l1))
    return jnp.where(iota == i1, p1, 0.0) + jnp.where(iota == i2, 1.0 - p1, 0.0)


_EB = 4


def _moe_kernel(x_ref, rw_ref, brw_ref, w_ref, bb_ref, out_ref, c_ref):
    g = pl.program_id(0)

    @pl.when(g == 0)
    def _init():
        x = x_ref[...]
        cw = _topk2_coeffs(jnp.dot(x, rw_ref[...], preferred_element_type=jnp.float32))
        cb = _topk2_coeffs(jnp.dot(x, brw_ref[...], preferred_element_type=jnp.float32))
        c_ref[...] = cw
        out_ref[...] = jnp.dot(cb, bb_ref[...], preferred_element_type=jnp.float32)

    c = c_ref[...]
    iota = jax.lax.broadcasted_iota(jnp.int32, c.shape, 1)
    acc = out_ref[...]
    xb = x_ref[...].astype(jnp.bfloat16)
    for j in range(_EB):
        e = g * _EB + j
        y = jnp.dot(xb, w_ref[j].astype(jnp.bfloat16),
                    preferred_element_type=jnp.float32)
        ce = jnp.sum(jnp.where(iota == e, c, 0.0), axis=1, keepdims=True)
        acc = acc + ce * y
    out_ref[...] = acc


def kernel(input_batch, router_w, bias_router_w, weight_bank, bias_bank):
    return pl.pallas_call(
        _moe_kernel,
        grid=(_E // _EB,),
        in_specs=[
            pl.BlockSpec((_B, _D), lambda e: (0, 0)),
            pl.BlockSpec((_D, _E), lambda e: (0, 0)),
            pl.BlockSpec((_D, _E), lambda e: (0, 0)),
            pl.BlockSpec((_EB, _D, _O), lambda e: (e, 0, 0)),
            pl.BlockSpec((_E, _O), lambda e: (0, 0)),
        ],
        out_specs=pl.BlockSpec((_B, _O), lambda e: (0, 0)),
        out_shape=jax.ShapeDtypeStruct((_B, _O), jnp.float32),
        scratch_shapes=[pltpu.VMEM((_B, _E), jnp.float32)],
        compiler_params=pltpu.CompilerParams(
            dimension_semantics=("arbitrary",),
        ),
    )(input_batch, router_w, bias_router_w, weight_bank, bias_bank)


# speedup vs baseline: 3.5561x; 1.0936x over previous
import jax
import jax.numpy as jnp
from jax.experimental import pallas as pl
from jax.experimental.pallas import tpu as pltpu

_B, _D, _O, _E = 256, 768, 256, 16


def _topk2_coeffs(logits):
    iota = jax.lax.broadcasted_iota(jnp.int32, logits.shape, 1)
    l1 = jnp.max(logits, axis=-1, keepdims=True)
    i1 = jnp.min(jnp.where(logits == l1, iota, _E), axis=-1, keepdims=True)
    masked = jnp.where(iota == i1, -jnp.inf, logits)
    l2 = jnp.max(masked, axis=-1, keepdims=True)
    i2 = jnp.min(jnp.where(masked == l2, iota, _E), axis=-1, keepdims=True)
    p1 = 1.0 / (1.0 + jnp.exp(l2 - l1))
    return jnp.where(iota == i1, p1, 0.0) + jnp.where(iota == i2, 1.0 - p1, 0.0)


_EB = 8


def _moe_kernel(x_ref, rw_ref, brw_ref, w_ref, bb_ref, out_ref, c_ref):
    g = pl.program_id(0)

    @pl.when(g == 0)
    def _init():
        x = x_ref[...]
        cw = _topk2_coeffs(jnp.dot(x, rw_ref[...], preferred_element_type=jnp.float32))
        cb = _topk2_coeffs(jnp.dot(x, brw_ref[...], preferred_element_type=jnp.float32))
        c_ref[...] = cw
        out_ref[...] = jnp.dot(cb, bb_ref[...], preferred_element_type=jnp.float32)

    c = c_ref[...]
    iota = jax.lax.broadcasted_iota(jnp.int32, c.shape, 1)
    acc = out_ref[...]
    x = x_ref[...]
    for j in range(_EB):
        e = g * _EB + j
        y = jnp.dot(x, w_ref[j], preferred_element_type=jnp.float32)
        ce = jnp.sum(jnp.where(iota == e, c, 0.0), axis=1, keepdims=True)
        acc = acc + ce * y
    out_ref[...] = acc


def kernel(input_batch, router_w, bias_router_w, weight_bank, bias_bank):
    return pl.pallas_call(
        _moe_kernel,
        grid=(_E // _EB,),
        in_specs=[
            pl.BlockSpec((_B, _D), lambda e: (0, 0)),
            pl.BlockSpec((_D, _E), lambda e: (0, 0)),
            pl.BlockSpec((_D, _E), lambda e: (0, 0)),
            pl.BlockSpec((_EB, _D, _O), lambda e: (e, 0, 0)),
            pl.BlockSpec((_E, _O), lambda e: (0, 0)),
        ],
        out_specs=pl.BlockSpec((_B, _O), lambda e: (0, 0)),
        out_shape=jax.ShapeDtypeStruct((_B, _O), jnp.float32),
        scratch_shapes=[pltpu.VMEM((_B, _E), jnp.float32)],
        compiler_params=pltpu.CompilerParams(
            dimension_semantics=("arbitrary",),
        ),
    )(input_batch, router_w, bias_router_w, weight_bank, bias_bank)
